# baseline (speedup 1.0000x reference)
import jax
import jax.numpy as jnp
from jax import lax
from jax.experimental import pallas as pl
from jax.experimental.pallas import tpu as pltpu

IN_DIM = 512
HIDDEN = 256
M = 3
NM = M + 1
H2 = (IN_DIM + HIDDEN) // 2


def _dot(a, b):
    return jnp.dot(a.astype(jnp.bfloat16), b.astype(jnp.bfloat16),
                   preferred_element_type=jnp.float32)


def _branch_body(feat_ref, adj_ref, att_ref, W1_ref, b1_ref, W2_ref, b2_ref,
                 W3_ref, b3_ref, Wq_ref, bq_ref, Wk_ref, bk_ref, beta_ref,
                 out_ref):
    feat = feat_ref[...]

    ps = []
    for m in range(NM):
        if m == 0:
            h = _dot(feat, W1_ref[0])
        else:
            x = att_ref[:, m - 1:m] * feat
            y = _dot(x, W1_ref[m])
            a = adj_ref[m - 1]
            s = jnp.sum(jnp.abs(a), axis=1, keepdims=True)
            s = jnp.where(s == 0.0, 1.0, s)
            h = _dot(a, y) / s
        h = jnp.maximum(h + b1_ref[m:m + 1, :], 0.0)
        h = jnp.maximum(_dot(h, W2_ref[m]) + b2_ref[m:m + 1, :], 0.0)
        p = _dot(h, W3_ref[m]) + b3_ref[m:m + 1, :]
        ps.append(p)

    Wq = Wq_ref[...]
    Wk = Wk_ref[...]
    bq = bq_ref[...]
    bk = bk_ref[...]
    Qs = [_dot(p, Wq) + bq for p in ps]
    Ks = [_dot(p, Wk) + bk for p in ps]

    scores = [[jnp.sum(Qs[m] * Ks[k], axis=1, keepdims=True)
               for k in range(NM)] for m in range(NM)]

    beta = beta_ref[...]
    for m in range(NM):
        mx = jnp.maximum(jnp.maximum(scores[m][0], scores[m][1]),
                         jnp.maximum(scores[m][2], scores[m][3]))
        es = [jnp.exp(scores[m][k] - mx) for k in range(NM)]
        den = es[0] + es[1] + es[2] + es[3]
        mix = (es[0] * ps[0] + es[1] * ps[1] + es[2] * ps[2] + es[3] * ps[3]) / den
        out_ref[:, m * HIDDEN:(m + 1) * HIDDEN] = beta * mix + ps[m]


def _dec_body(dr_ref, ds_ref, Wdec_ref, bdec_ref, wa_ref, out_ref):
    w = wa_ref[...]
    e = jnp.exp(w - jnp.max(w))
    w = e / jnp.sum(e)

    cols = []
    for m in range(NM):
        dt = _dot(ds_ref[m], Wdec_ref[m]) + bdec_ref[m:m + 1, :]
        cols.append(dt * w[0:1, m:m + 1])
    B = jnp.concatenate(cols, axis=1)
    A = jnp.concatenate([dr_ref[0], dr_ref[1], dr_ref[2], dr_ref[3]],
                        axis=1)
    out_ref[...] = lax.dot_general(
        A.astype(jnp.bfloat16), B.astype(jnp.bfloat16),
        (((1,), (1,)), ((), ())), preferred_element_type=jnp.float32)


def _branch(feat, adj, att, W1, b1, W2, b2, W3, b3, Wq, bq, Wk, bk, beta):
    n = feat.shape[0]
    att_t = jnp.transpose(att[:, :, 0])
    out2 = pl.pallas_call(
        _branch_body,
        out_shape=jax.ShapeDtypeStruct((n, NM * HIDDEN), jnp.float32),
        compiler_params=pltpu.CompilerParams(
            vmem_limit_bytes=128 * 1024 * 1024),
    )(feat, adj, att_t, W1, b1, W2, b2, W3, b3,
      Wq, bq.reshape(1, HIDDEN), Wk, bk.reshape(1, HIDDEN),
      beta.reshape(1, 1))
    return out2.reshape(NM, n, HIDDEN)


def kernel(drug_feat, disease_feat, adj_drug, adj_disease, att_drug,
           att_disease, W1d, b1d, W2d, b2d, W3d, b3d, Wqd, bqd, Wkd, bkd,
           Wvd, bvd, betad, W1s, b1s, W2s, b2s, W3s, b3s, Wqs, bqs, Wks, bks,
           Wvs, bvs, betas, weight_attn, Wdec, bdec):
    dr = _branch(drug_feat, adj_drug, att_drug, W1d, b1d, W2d, b2d, W3d, b3d,
                 Wqd, bqd, Wkd, bkd, betad)
    ds = _branch(disease_feat, adj_disease, att_disease, W1s, b1s, W2s, b2s,
                 W3s, b3s, Wqs, bqs, Wks, bks, betas)
    n_drug = drug_feat.shape[0]
    n_dis = disease_feat.shape[0]
    out = pl.pallas_call(
        _dec_body,
        out_shape=jax.ShapeDtypeStruct((n_drug, n_dis), jnp.float32),
        compiler_params=pltpu.CompilerParams(
            vmem_limit_bytes=128 * 1024 * 1024),
    )(dr, ds, Wdec, bdec, weight_attn.reshape(1, NM))
    return out


# speedup vs baseline: 1.0027x; 1.0027x over previous
import jax
import jax.numpy as jnp
from jax import lax
from jax.experimental import pallas as pl
from jax.experimental.pallas import tpu as pltpu

IN_DIM = 512
HIDDEN = 256
M = 3
NM = M + 1
H2 = (IN_DIM + HIDDEN) // 2


def _dot(a, b):
    return jnp.dot(a, b, preferred_element_type=jnp.float32)


def _branch_body(feat_ref, adj_ref, att_ref, W1_ref, b1_ref, W2_ref, b2_ref,
                 W3_ref, b3_ref, Wq_ref, bq_ref, Wk_ref, bk_ref, beta_ref,
                 out_ref):
    feat = feat_ref[...]

    ps = []
    for m in range(NM):
        if m == 0:
            h = _dot(feat, W1_ref[0])
        else:
            x = att_ref[:, m - 1:m] * feat
            y = _dot(x, W1_ref[m])
            a = adj_ref[m - 1]
            s = jnp.sum(jnp.abs(a), axis=1, keepdims=True)
            s = jnp.where(s == 0.0, 1.0, s)
            h = _dot(a, y) / s
        h = jnp.maximum(h + b1_ref[m:m + 1, :], 0.0)
        h = jnp.maximum(_dot(h, W2_ref[m]) + b2_ref[m:m + 1, :], 0.0)
        p = _dot(h, W3_ref[m]) + b3_ref[m:m + 1, :]
        ps.append(p)

    Wq = Wq_ref[...]
    Wk = Wk_ref[...]
    bq = bq_ref[...]
    bk = bk_ref[...]
    Qs = [_dot(p, Wq) + bq for p in ps]
    Ks = [_dot(p, Wk) + bk for p in ps]

    scores = [[jnp.sum(Qs[m] * Ks[k], axis=1, keepdims=True)
               for k in range(NM)] for m in range(NM)]

    beta = beta_ref[...]
    for m in range(NM):
        mx = jnp.maximum(jnp.maximum(scores[m][0], scores[m][1]),
                         jnp.maximum(scores[m][2], scores[m][3]))
        es = [jnp.exp(scores[m][k] - mx) for k in range(NM)]
        den = es[0] + es[1] + es[2] + es[3]
        mix = (es[0] * ps[0] + es[1] * ps[1] + es[2] * ps[2] + es[3] * ps[3]) / den
        out_ref[:, m * HIDDEN:(m + 1) * HIDDEN] = beta * mix + ps[m]


def _dec_body(dr_ref, ds_ref, Wdec_ref, bdec_ref, wa_ref, out_ref):
    w = wa_ref[...]
    e = jnp.exp(w - jnp.max(w))
    w = e / jnp.sum(e)

    cols = []
    for m in range(NM):
        dt = _dot(ds_ref[m], Wdec_ref[m]) + bdec_ref[m:m + 1, :]
        cols.append(dt * w[0:1, m:m + 1])
    B = jnp.concatenate(cols, axis=1)
    A = jnp.concatenate([dr_ref[0], dr_ref[1], dr_ref[2], dr_ref[3]],
                        axis=1)
    out_ref[...] = lax.dot_general(
        A, B, (((1,), (1,)), ((), ())), preferred_element_type=jnp.float32)


def _branch(feat, adj, att, W1, b1, W2, b2, W3, b3, Wq, bq, Wk, bk, beta):
    n = feat.shape[0]
    att_t = jnp.transpose(att[:, :, 0])
    out2 = pl.pallas_call(
        _branch_body,
        out_shape=jax.ShapeDtypeStruct((n, NM * HIDDEN), jnp.float32),
        compiler_params=pltpu.CompilerParams(
            vmem_limit_bytes=128 * 1024 * 1024),
    )(feat, adj, att_t, W1, b1, W2, b2, W3, b3,
      Wq, bq.reshape(1, HIDDEN), Wk, bk.reshape(1, HIDDEN),
      beta.reshape(1, 1))
    return out2.reshape(NM, n, HIDDEN)


def kernel(drug_feat, disease_feat, adj_drug, adj_disease, att_drug,
           att_disease, W1d, b1d, W2d, b2d, W3d, b3d, Wqd, bqd, Wkd, bkd,
           Wvd, bvd, betad, W1s, b1s, W2s, b2s, W3s, b3s, Wqs, bqs, Wks, bks,
           Wvs, bvs, betas, weight_attn, Wdec, bdec):
    dr = _branch(drug_feat, adj_drug, att_drug, W1d, b1d, W2d, b2d, W3d, b3d,
                 Wqd, bqd, Wkd, bkd, betad)
    ds = _branch(disease_feat, adj_disease, att_disease, W1s, b1s, W2s, b2s,
                 W3s, b3s, Wqs, bqs, Wks, bks, betas)
    n_drug = drug_feat.shape[0]
    n_dis = disease_feat.shape[0]
    out = pl.pallas_call(
        _dec_body,
        out_shape=jax.ShapeDtypeStruct((n_drug, n_dis), jnp.float32),
        compiler_params=pltpu.CompilerParams(
            vmem_limit_bytes=128 * 1024 * 1024),
    )(dr, ds, Wdec, bdec, weight_attn.reshape(1, NM))
    return out
